# baseline (speedup 1.0000x reference)
import functools

import jax
import jax.numpy as jnp
from jax import lax
from jax.experimental import pallas as pl
from jax.experimental.pallas import tpu as pltpu
from jax.experimental.pallas import tpu_sc as plsc

N = 10000
F = 128
NCLASS = 2
NC, NS, L = 2, 16, 16
NW = NC * NS
K = 64
SB_CH = 32
BLK_E = SB_CH * K
A_BLK = 2
B_BLK = 8
NV = NS * (A_BLK + B_BLK)
E_PAD = NV * BLK_E
ROWS_PER_TILE = 640
N_PAD = NS * ROWS_PER_TILE
DEG_W = 8
NBUF = 4

_mesh = plsc.VectorSubcoreMesh(
    core_axis_name="c", subcore_axis_name="s", num_cores=NC, num_subcores=NS)


@functools.partial(
    pl.kernel,
    out_type=jax.ShapeDtypeStruct((NC, N_PAD, DEG_W), jnp.float32),
    mesh=_mesh,
    scratch_types=[
        pltpu.VMEM((SB_CH, K), jnp.int32),
        pltpu.VMEM((K, DEG_W), jnp.float32),
        pltpu.VMEM((K, DEG_W), jnp.float32),
        pltpu.VMEM_SHARED((N_PAD, DEG_W), jnp.float32),
        pltpu.SemaphoreType.DMA,
    ],
)
def _deg_kernel(col_hbm, out_hbm, cidx, ones_v, zero_v, deg_sh, sem_s):
    c = lax.axis_index("c")
    s = lax.axis_index("s")

    def fill(i, carry):
        ones_v[i] = jnp.ones((DEG_W,), jnp.float32)
        zero_v[i] = jnp.zeros((DEG_W,), jnp.float32)
        return carry
    lax.fori_loop(0, K, fill, 0)

    base = s * ROWS_PER_TILE
    for p in range(ROWS_PER_TILE // K):
        pltpu.sync_copy(zero_v, deg_sh.at[pl.ds(base + p * K, K)])
    plsc.subcore_barrier()

    nv = jnp.where(c == 0, A_BLK, B_BLK)
    v0 = jnp.where(c == 0, s * A_BLK, NS * A_BLK + s * B_BLK)

    def sblock(p, carry):
        pltpu.sync_copy(col_hbm.at[v0 + p], cidx)

        def scat(q, c2):
            pltpu.async_copy(ones_v, deg_sh.at[cidx.at[q]], sem_s, add=True)
            return c2
        lax.fori_loop(0, SB_CH, scat, 0)

        def drain(q, c2):
            pltpu.make_async_copy(ones_v, deg_sh.at[cidx.at[q]], sem_s).wait()
            return c2
        lax.fori_loop(0, SB_CH, drain, 0)
        return carry
    lax.fori_loop(0, nv, sblock, 0)

    plsc.subcore_barrier()
    pltpu.sync_copy(deg_sh.at[pl.ds(base, ROWS_PER_TILE)],
                    out_hbm.at[c, pl.ds(base, ROWS_PER_TILE)])


@functools.partial(
    pl.kernel,
    out_type=jax.ShapeDtypeStruct((NC, N_PAD, F), jnp.float32),
    mesh=_mesh,
    scratch_types=[
        pltpu.VMEM((SB_CH, K), jnp.int32),
        pltpu.VMEM((SB_CH, K), jnp.int32),
        *[pltpu.VMEM((K, F), jnp.float32) for _ in range(NBUF)],
        pltpu.VMEM_SHARED((N_PAD, F), jnp.float32),
        *[pltpu.SemaphoreType.DMA for _ in range(2 * NBUF)],
    ],
)
def _edge_kernel(row_hbm, col_hbm, g_hbm, out_hbm, ridx, cidx, *rest):
    bufs = rest[:NBUF]
    acc_sh = rest[NBUF]
    gsem = rest[NBUF + 1:NBUF + 1 + NBUF]
    ssem = rest[NBUF + 1 + NBUF:]
    c = lax.axis_index("c")
    s = lax.axis_index("s")

    def zrow(i, carry):
        for kk in range(F // L):
            bufs[0][i, pl.ds(kk * L, L)] = jnp.zeros((L,), jnp.float32)
        return carry
    lax.fori_loop(0, K, zrow, 0)
    base = s * ROWS_PER_TILE
    for p in range(ROWS_PER_TILE // K):
        pltpu.sync_copy(bufs[0], acc_sh.at[pl.ds(base + p * K, K)])
    plsc.subcore_barrier()

    nv = jnp.where(c == 0, A_BLK, B_BLK)
    v0 = jnp.where(c == 0, s * A_BLK, NS * A_BLK + s * B_BLK)

    def sblock(p, carry):
        pltpu.sync_copy(row_hbm.at[v0 + p], ridx)
        pltpu.sync_copy(col_hbm.at[v0 + p], cidx)
        for b in range(NBUF):
            pltpu.async_copy(g_hbm.at[ridx.at[b]], bufs[b], gsem[b])

        def body(i, c2):
            q0 = i * NBUF
            for b in range(NBUF):
                q = q0 + b
                pltpu.make_async_copy(
                    g_hbm.at[ridx.at[q]], bufs[b], gsem[b]).wait()
                pltpu.async_copy(
                    bufs[b], acc_sh.at[cidx.at[q]], ssem[b], add=True)

                @pl.when(q + NBUF < SB_CH)
                def _():
                    pltpu.make_async_copy(
                        bufs[b], acc_sh.at[cidx.at[q]], ssem[b]).wait()
                    pltpu.async_copy(
                        g_hbm.at[ridx.at[q + NBUF]], bufs[b], gsem[b])
            return c2
        lax.fori_loop(0, SB_CH // NBUF, body, 0)
        for b in range(NBUF):
            pltpu.make_async_copy(
                bufs[b], acc_sh.at[cidx.at[SB_CH - NBUF + b]], ssem[b]).wait()
        return carry
    lax.fori_loop(0, nv, sblock, 0)

    plsc.subcore_barrier()
    pltpu.sync_copy(acc_sh.at[pl.ds(base, ROWS_PER_TILE)],
                    out_hbm.at[c, pl.ds(base, ROWS_PER_TILE)])


def _prep_body(x_ref, w_ref, degp_ref, g_ref):
    deg = degp_ref[0, :, 0:1] + degp_ref[1, :, 0:1] + 1.0
    dis = lax.rsqrt(deg)
    h = jnp.dot(x_ref[...], w_ref[...], preferred_element_type=jnp.float32)
    g_ref[pl.ds(0, N), :] = h * dis[0:N]
    g_ref[pl.ds(N, N_PAD - N), :] = jnp.zeros((N_PAD - N, F), jnp.float32)


def _final_body(accp_ref, g_ref, degp_ref, b_ref, gam_ref, bet_ref,
                fcw_ref, fcb_ref, logits_ref, embed_ref):
    deg = degp_ref[0, :, 0:1] + degp_ref[1, :, 0:1] + 1.0
    dis = lax.rsqrt(deg[0:N])
    ssum = accp_ref[0, 0:N, :] + accp_ref[1, 0:N, :] + g_ref[0:N, :]
    out = ssum * dis + b_ref[...]
    mean = jnp.mean(out, axis=0, keepdims=True)
    cent = out - mean
    var = jnp.mean(cent * cent, axis=0, keepdims=True)
    embed = cent * lax.rsqrt(var + 1e-5) * gam_ref[...] + bet_ref[...]
    embed_ref[...] = embed
    logits_ref[...] = jnp.dot(embed[:, 0:F // 2], fcw_ref[...],
                              preferred_element_type=jnp.float32) + fcb_ref[...]


def kernel(x, edge_index, W_gc, b_gc, bn_gamma, bn_beta, fc_W, fc_b):
    E = edge_index.shape[1]
    row = edge_index[0].astype(jnp.int32)
    col = edge_index[1].astype(jnp.int32)
    padv = jnp.full((E_PAD - E,), N, jnp.int32)
    row_t = jnp.concatenate([row, padv]).reshape(NV, SB_CH, K)
    col_t = jnp.concatenate([col, padv]).reshape(NV, SB_CH, K)

    degp = _deg_kernel(col_t)

    g_pad = pl.pallas_call(
        _prep_body,
        out_shape=jax.ShapeDtypeStruct((N_PAD, F), jnp.float32),
    )(x, W_gc, degp)

    accp = _edge_kernel(row_t, col_t, g_pad)

    logits, embed = pl.pallas_call(
        _final_body,
        out_shape=[
            jax.ShapeDtypeStruct((N, NCLASS), jnp.float32),
            jax.ShapeDtypeStruct((N, F), jnp.float32),
        ],
    )(accp, g_pad, degp, b_gc.reshape(1, F), bn_gamma.reshape(1, F),
      bn_beta.reshape(1, F), fc_W, fc_b.reshape(1, NCLASS))
    return (logits, embed)


# speedup vs baseline: 1.2162x; 1.2162x over previous
import functools

import jax
import jax.numpy as jnp
from jax import lax
from jax.experimental import pallas as pl
from jax.experimental.pallas import tpu as pltpu
from jax.experimental.pallas import tpu_sc as plsc

N = 10000
F = 128
NCLASS = 2
NC, NS, L = 2, 16, 16
NW = NC * NS
K = 64
SB_CH = 64
BLK_E = SB_CH * K
A_BLK = 5
B_BLK = 0
NV = NS * (A_BLK + B_BLK)
E_PAD = NV * BLK_E
ROWS_PER_TILE = 640
N_PAD = NS * ROWS_PER_TILE
DEG_W = 8
NBUF = 4

_mesh = plsc.VectorSubcoreMesh(
    core_axis_name="c", subcore_axis_name="s", num_cores=NC, num_subcores=NS)


@functools.partial(
    pl.kernel,
    out_type=jax.ShapeDtypeStruct((NC, N_PAD, DEG_W), jnp.float32),
    mesh=_mesh,
    scratch_types=[
        pltpu.VMEM((SB_CH, K), jnp.int32),
        pltpu.VMEM((K, DEG_W), jnp.float32),
        pltpu.VMEM_SHARED((N_PAD, DEG_W), jnp.float32),
        pltpu.SemaphoreType.DMA,
    ],
)
def _deg_kernel(col_hbm, out_hbm, cidx, ones_v, deg_sh, sem_s):
    c = lax.axis_index("c")
    s = lax.axis_index("s")

    def fill0(i, carry):
        ones_v[i] = jnp.zeros((DEG_W,), jnp.float32)
        return carry
    lax.fori_loop(0, K, fill0, 0)

    base = s * ROWS_PER_TILE
    for p in range(ROWS_PER_TILE // K):
        pltpu.sync_copy(ones_v, deg_sh.at[pl.ds(base + p * K, K)])

    def fill1(i, carry):
        ones_v[i] = jnp.ones((DEG_W,), jnp.float32)
        return carry
    lax.fori_loop(0, K, fill1, 0)
    plsc.subcore_barrier()

    nv = jnp.where(c == 0, A_BLK, B_BLK)
    v0 = jnp.where(c == 0, s * A_BLK, NS * A_BLK + s * B_BLK)

    def sblock(p, carry):
        pltpu.sync_copy(col_hbm.at[v0 + p], cidx)

        def scat(q, c2):
            pltpu.async_copy(ones_v, deg_sh.at[cidx.at[q]], sem_s, add=True)
            return c2
        lax.fori_loop(0, SB_CH, scat, 0)

        def drain(q, c2):
            pltpu.make_async_copy(ones_v, deg_sh.at[cidx.at[q]], sem_s).wait()
            return c2
        lax.fori_loop(0, SB_CH, drain, 0)
        return carry
    lax.fori_loop(0, nv, sblock, 0)

    plsc.subcore_barrier()
    pltpu.sync_copy(deg_sh.at[pl.ds(base, ROWS_PER_TILE)],
                    out_hbm.at[c, pl.ds(base, ROWS_PER_TILE)])


@functools.partial(
    pl.kernel,
    out_type=jax.ShapeDtypeStruct((NC, N_PAD, F), jnp.float32),
    mesh=_mesh,
    scratch_types=[
        pltpu.VMEM((SB_CH, K), jnp.int32),
        pltpu.VMEM((SB_CH, K), jnp.int32),
        *[pltpu.VMEM((K, F), jnp.float32) for _ in range(NBUF)],
        pltpu.VMEM_SHARED((N_PAD, F), jnp.float32),
        *[pltpu.SemaphoreType.DMA for _ in range(2 * NBUF)],
    ],
)
def _edge_kernel(row_hbm, col_hbm, g_hbm, out_hbm, ridx, cidx, *rest):
    bufs = rest[:NBUF]
    acc_sh = rest[NBUF]
    gsem = rest[NBUF + 1:NBUF + 1 + NBUF]
    ssem = rest[NBUF + 1 + NBUF:]
    c = lax.axis_index("c")
    s = lax.axis_index("s")

    def zrow(i, carry):
        for kk in range(F // L):
            bufs[0][i, pl.ds(kk * L, L)] = jnp.zeros((L,), jnp.float32)
        return carry
    lax.fori_loop(0, K, zrow, 0)
    base = s * ROWS_PER_TILE
    for p in range(ROWS_PER_TILE // K):
        pltpu.sync_copy(bufs[0], acc_sh.at[pl.ds(base + p * K, K)])
    plsc.subcore_barrier()

    nv = jnp.where(c == 0, A_BLK, B_BLK)
    v0 = jnp.where(c == 0, s * A_BLK, NS * A_BLK + s * B_BLK)

    def sblock(p, carry):
        pltpu.sync_copy(row_hbm.at[v0 + p], ridx)
        pltpu.sync_copy(col_hbm.at[v0 + p], cidx)
        for b in range(NBUF):
            pltpu.async_copy(g_hbm.at[ridx.at[b]], bufs[b], gsem[b])

        def body(i, c2):
            q0 = i * NBUF
            for b in range(NBUF):
                q = q0 + b
                pltpu.make_async_copy(
                    g_hbm.at[ridx.at[q]], bufs[b], gsem[b]).wait()
                pltpu.async_copy(
                    bufs[b], acc_sh.at[cidx.at[q]], ssem[b], add=True)

                @pl.when(q + NBUF < SB_CH)
                def _():
                    pltpu.make_async_copy(
                        bufs[b], acc_sh.at[cidx.at[q]], ssem[b]).wait()
                    pltpu.async_copy(
                        g_hbm.at[ridx.at[q + NBUF]], bufs[b], gsem[b])
            return c2
        lax.fori_loop(0, SB_CH // NBUF, body, 0)
        for b in range(NBUF):
            pltpu.make_async_copy(
                bufs[b], acc_sh.at[cidx.at[SB_CH - NBUF + b]], ssem[b]).wait()
        return carry
    lax.fori_loop(0, nv, sblock, 0)

    plsc.subcore_barrier()
    pltpu.sync_copy(acc_sh.at[pl.ds(base, ROWS_PER_TILE)],
                    out_hbm.at[c, pl.ds(base, ROWS_PER_TILE)])


def _mm_body(x_ref, w_ref, h_ref):
    h_ref[...] = jnp.dot(x_ref[...], w_ref[...],
                         preferred_element_type=jnp.float32)


def _scale_body(h_ref, degp_ref, g_ref):
    deg = degp_ref[0, :, 0:1] + degp_ref[1, :, 0:1] + 1.0
    dis = lax.rsqrt(deg)
    g_ref[pl.ds(0, N), :] = h_ref[...] * dis[0:N]
    g_ref[pl.ds(N, N_PAD - N), :] = jnp.zeros((N_PAD - N, F), jnp.float32)


def _final_body(accp_ref, g_ref, degp_ref, b_ref, gam_ref, bet_ref,
                fcw_ref, fcb_ref, logits_ref, embed_ref):
    deg = degp_ref[0, :, 0:1] + degp_ref[1, :, 0:1] + 1.0
    dis = lax.rsqrt(deg[0:N])
    ssum = accp_ref[0, 0:N, :] + accp_ref[1, 0:N, :] + g_ref[0:N, :]
    out = ssum * dis + b_ref[...]
    mean = jnp.mean(out, axis=0, keepdims=True)
    cent = out - mean
    var = jnp.mean(cent * cent, axis=0, keepdims=True)
    embed = cent * lax.rsqrt(var + 1e-5) * gam_ref[...] + bet_ref[...]
    embed_ref[...] = embed
    logits_ref[...] = jnp.dot(embed[:, 0:F // 2], fcw_ref[...],
                              preferred_element_type=jnp.float32) + fcb_ref[...]


def kernel(x, edge_index, W_gc, b_gc, bn_gamma, bn_beta, fc_W, fc_b):
    E = edge_index.shape[1]
    row = edge_index[0].astype(jnp.int32)
    col = edge_index[1].astype(jnp.int32)
    padv = jnp.full((E_PAD - E,), N, jnp.int32)
    row_t = jnp.concatenate([row, padv]).reshape(NV, SB_CH, K)
    col_t = jnp.concatenate([col, padv]).reshape(NV, SB_CH, K)

    degp = _deg_kernel(col_t)

    h = pl.pallas_call(
        _mm_body,
        out_shape=jax.ShapeDtypeStruct((N, F), jnp.float32),
    )(x, W_gc)

    g_pad = pl.pallas_call(
        _scale_body,
        out_shape=jax.ShapeDtypeStruct((N_PAD, F), jnp.float32),
    )(h, degp)

    accp = _edge_kernel(row_t, col_t, g_pad)

    logits, embed = pl.pallas_call(
        _final_body,
        out_shape=[
            jax.ShapeDtypeStruct((N, NCLASS), jnp.float32),
            jax.ShapeDtypeStruct((N, F), jnp.float32),
        ],
    )(accp, g_pad, degp, b_gc.reshape(1, F), bn_gamma.reshape(1, F),
      bn_beta.reshape(1, F), fc_W, fc_b.reshape(1, NCLASS))
    return (logits, embed)
